# Initial kernel scaffold; baseline (speedup 1.0000x reference)
import jax
import jax.numpy as jnp
from jax.experimental import pallas as pl


def kernel(x, edge_index, batch, W1, b1, W2, b2, Wfc, bfc):
    raise NotImplementedError("write your pallas kernel here")


# speedup vs baseline: 9.3245x; 9.3245x over previous
import functools

import jax
import jax.numpy as jnp
from jax import lax
from jax.experimental import pallas as pl
from jax.experimental.pallas import tpu as pltpu
from jax.experimental.pallas import tpu_sc as plsc

N_NODES = 10000
N_EDGES = 320000
D = 128
G = 128

NC = 2
NS = 16
NW = NC * NS
CHUNK = 128
EPT = 10240
NCHUNK = EPT // CHUNK
E_PAD = EPT * NW
N_PAD = 10240
SLAB = N_PAD // NS


def _sc_deg_body(dst_hbm, ones_hbm, zeros_hbm, out_hbm, dst_v, ones_v, acc_sh, sem):
    c = lax.axis_index("c")
    s = lax.axis_index("s")
    wid = c * NS + s
    slab = s * SLAB
    pltpu.sync_copy(zeros_hbm, acc_sh.at[pl.ds(slab, SLAB)])
    pltpu.sync_copy(dst_hbm.at[wid], dst_v)
    pltpu.sync_copy(ones_hbm, ones_v)
    plsc.subcore_barrier()

    def body(j, carry):
        pltpu.sync_copy(ones_v, acc_sh.at[dst_v.at[j]], add=True)
        return carry

    lax.fori_loop(0, NCHUNK, body, 0)
    plsc.subcore_barrier()
    pltpu.sync_copy(acc_sh.at[pl.ds(slab, SLAB)],
                    out_hbm.at[c].at[pl.ds(slab, SLAB)])


_sc_deg = functools.partial(
    pl.kernel,
    out_type=jax.ShapeDtypeStruct((NC, N_PAD), jnp.float32),
    mesh=plsc.VectorSubcoreMesh(core_axis_name="c", subcore_axis_name="s"),
    scratch_types=[
        pltpu.VMEM((NCHUNK, CHUNK), jnp.int32),
        pltpu.VMEM((CHUNK,), jnp.float32),
        pltpu.VMEM_SHARED((N_PAD,), jnp.float32),
        pltpu.SemaphoreType.DMA,
    ],
)(_sc_deg_body)


def _sc_agg_body(hp_hbm, src_hbm, dst_hbm, zeros_hbm, out_hbm,
                 src_v, dst_v, rows_v, acc_sh, sem):
    c = lax.axis_index("c")
    s = lax.axis_index("s")
    wid = c * NS + s
    slab = s * SLAB
    pltpu.sync_copy(zeros_hbm, acc_sh.at[pl.ds(slab, SLAB)])
    pltpu.sync_copy(src_hbm.at[wid], src_v)
    pltpu.sync_copy(dst_hbm.at[wid], dst_v)
    plsc.subcore_barrier()

    def body(j, carry):
        pltpu.async_copy(hp_hbm.at[src_v.at[j]], rows_v, sem).wait()
        pltpu.sync_copy(rows_v, acc_sh.at[dst_v.at[j]], add=True)
        return carry

    lax.fori_loop(0, NCHUNK, body, 0)
    plsc.subcore_barrier()
    pltpu.sync_copy(acc_sh.at[pl.ds(slab, SLAB)],
                    out_hbm.at[c].at[pl.ds(slab, SLAB)])


_sc_agg = functools.partial(
    pl.kernel,
    out_type=jax.ShapeDtypeStruct((NC, N_PAD, D), jnp.float32),
    mesh=plsc.VectorSubcoreMesh(core_axis_name="c", subcore_axis_name="s"),
    scratch_types=[
        pltpu.VMEM((NCHUNK, CHUNK), jnp.int32),
        pltpu.VMEM((NCHUNK, CHUNK), jnp.int32),
        pltpu.VMEM((CHUNK, D), jnp.float32),
        pltpu.VMEM_SHARED((N_PAD, D), jnp.float32),
        pltpu.SemaphoreType.DMA,
    ],
)(_sc_agg_body)


RB = 1000
NRB = N_NODES // RB


def _tc_prep1_body(x_ref, w1_ref, d0_ref, d1_ref, hp1_ref, dinv_ref):
    deg = d0_ref[0, 0, :] + d1_ref[0, 0, :] + 1.0
    dv = lax.rsqrt(deg).reshape(RB, 1)
    h = jnp.dot(x_ref[...], w1_ref[...], preferred_element_type=jnp.float32)
    hp1_ref[...] = h * dv
    dinv_ref[...] = dv


def _tc_mid_body(a0_ref, a1_ref, hp_ref, dinv_ref, b_ref, w2_ref, hp2_ref):
    dv = dinv_ref[...]
    h = dv * (a0_ref[...] + a1_ref[...] + hp_ref[...]) + b_ref[...]
    h = jnp.maximum(h, 0.0)
    hp2_ref[...] = jnp.dot(h, w2_ref[...], preferred_element_type=jnp.float32) * dv


def _tc_final_body(a0_ref, a1_ref, hp_ref, dinv_ref, b_ref, batch_ref,
                   wfc_ref, bfc_ref, out_ref, acc, cnt):
    i = pl.program_id(0)

    @pl.when(i == 0)
    def _():
        acc[...] = jnp.zeros_like(acc)
        cnt[...] = jnp.zeros_like(cnt)

    dv = dinv_ref[...]
    h = dv * (a0_ref[...] + a1_ref[...] + hp_ref[...]) + b_ref[...]
    h = jnp.maximum(h, 0.0)
    bb = batch_ref[0, 0, :]
    gids = lax.broadcasted_iota(jnp.int32, (G, RB), 0)
    onehot = (gids == bb[None, :]).astype(jnp.float32)
    acc[...] += jnp.dot(onehot, h, preferred_element_type=jnp.float32)
    cnt[...] += jnp.sum(onehot, axis=1, keepdims=True)

    @pl.when(i == NRB - 1)
    def _():
        emb = acc[...] / jnp.maximum(cnt[...], 1.0)
        out_ref[...] = (
            jnp.dot(emb, wfc_ref[...], preferred_element_type=jnp.float32)
            + bfc_ref[...]
        )


def kernel(x, edge_index, batch, W1, b1, W2, b2, Wfc, bfc):
    src = edge_index[0].astype(jnp.int32)
    dst = edge_index[1].astype(jnp.int32)
    npad = E_PAD - N_EDGES
    srcp = jnp.concatenate([src, jnp.zeros((npad,), jnp.int32)]).reshape(NW, NCHUNK, CHUNK)
    dstp = jnp.concatenate([dst, jnp.full((npad,), N_NODES, jnp.int32)]).reshape(NW, NCHUNK, CHUNK)
    ones_c = jnp.ones((CHUNK,), jnp.float32)
    zeros_1d = jnp.zeros((SLAB,), jnp.float32)
    zeros_2d = jnp.zeros((SLAB, D), jnp.float32)

    degp = _sc_deg(dstp, ones_c, zeros_1d)
    d0 = degp[0, :N_NODES].reshape(NRB, 1, RB)
    d1 = degp[1, :N_NODES].reshape(NRB, 1, RB)

    row_bs = pl.BlockSpec((RB, D), lambda i: (i, 0))
    deg_bs = pl.BlockSpec((1, 1, RB), lambda i: (i, 0, 0))
    dinv_bs = pl.BlockSpec((RB, 1), lambda i: (i, 0))
    full_bs = pl.BlockSpec((D, D), lambda i: (0, 0))
    bias_bs = pl.BlockSpec((1, D), lambda i: (0, 0))

    hp1, dinv = pl.pallas_call(
        _tc_prep1_body,
        grid=(NRB,),
        in_specs=[row_bs, full_bs, deg_bs, deg_bs],
        out_specs=[row_bs, dinv_bs],
        out_shape=[
            jax.ShapeDtypeStruct((N_NODES, D), jnp.float32),
            jax.ShapeDtypeStruct((N_NODES, 1), jnp.float32),
        ],
    )(x, W1, d0, d1)

    agg1 = _sc_agg(hp1, srcp, dstp, zeros_2d)

    hp2 = pl.pallas_call(
        _tc_mid_body,
        grid=(NRB,),
        in_specs=[row_bs, row_bs, row_bs, dinv_bs, bias_bs, full_bs],
        out_specs=row_bs,
        out_shape=jax.ShapeDtypeStruct((N_NODES, D), jnp.float32),
    )(agg1[0, :N_NODES], agg1[1, :N_NODES], hp1, dinv, b1.reshape(1, D), W2)

    agg2 = _sc_agg(hp2, srcp, dstp, zeros_2d)

    batchr = batch.astype(jnp.int32).reshape(NRB, 1, RB)
    wfc_pad = jnp.zeros((D, D), jnp.float32).at[:, : Wfc.shape[1]].set(Wfc)
    bfc_pad = jnp.zeros((1, D), jnp.float32).at[0, : bfc.shape[0]].set(bfc)

    out_pad = pl.pallas_call(
        _tc_final_body,
        grid=(NRB,),
        in_specs=[row_bs, row_bs, row_bs, dinv_bs, bias_bs, deg_bs,
                  full_bs, bias_bs],
        out_specs=pl.BlockSpec((G, D), lambda i: (0, 0)),
        out_shape=jax.ShapeDtypeStruct((G, D), jnp.float32),
        scratch_shapes=[
            pltpu.VMEM((G, D), jnp.float32),
            pltpu.VMEM((G, 1), jnp.float32),
        ],
    )(agg2[0, :N_NODES], agg2[1, :N_NODES], hp2, dinv, b2.reshape(1, D),
      batchr, wfc_pad, bfc_pad)

    return out_pad[:, : Wfc.shape[1]]
